# Initial kernel scaffold; baseline (speedup 1.0000x reference)
import jax
import jax.numpy as jnp
from jax.experimental import pallas as pl


def kernel(edges, nodes, globals_, receivers, senders, W, b):
    raise NotImplementedError("write your pallas kernel here")


# speedup vs baseline: 5.4622x; 5.4622x over previous
import functools

import jax
import jax.numpy as jnp
from jax import lax
from jax.experimental import pallas as pl
from jax.experimental.pallas import tpu as pltpu
from jax.experimental.pallas import tpu_sc as plsc

_NC = 2
_NS = 16
_NW = _NC * _NS
_CHUNK = 2000


def _proj_body(nodes_ref, wcat_ref, glob_ref, wg_ref, b_ref,
               pr_ref, ps_ref, c_ref):
    p = jnp.dot(nodes_ref[...], wcat_ref[...],
                preferred_element_type=jnp.float32)
    pr_ref[...] = p[:, :16]
    ps_ref[...] = p[:, 16:]
    c_ref[...] = jnp.dot(glob_ref[...], wg_ref[...],
                         preferred_element_type=jnp.float32) + b_ref[...]


def _sc_gather_body(epw, nchunk, pr_hbm, ps_hbm, recv_hbm, send_hbm,
                    gr_hbm, gs_hbm, ridx, sidx, rrows, srows, sem_r, sem_s):
    wid = lax.axis_index("s") * _NC + lax.axis_index("c")
    base = wid * epw

    def body(ci, carry):
        off = base + ci * _CHUNK
        pltpu.sync_copy(recv_hbm.at[pl.ds(off, _CHUNK)], ridx)
        pltpu.sync_copy(send_hbm.at[pl.ds(off, _CHUNK)], sidx)
        cr = pltpu.async_copy(pr_hbm.at[ridx], rrows, sem_r)
        cs = pltpu.async_copy(ps_hbm.at[sidx], srows, sem_s)
        cr.wait()
        cs.wait()
        pltpu.sync_copy(rrows, gr_hbm.at[pl.ds(off, _CHUNK)])
        pltpu.sync_copy(srows, gs_hbm.at[pl.ds(off, _CHUNK)])
        return carry

    lax.fori_loop(0, nchunk, body, 0)


def _combine_body(e_ref, gr_ref, gs_ref, wbig_ref, cbig_ref, o_ref):
    o_ref[...] = (jnp.dot(e_ref[...], wbig_ref[...],
                          preferred_element_type=jnp.float32)
                  + gr_ref[...] + gs_ref[...] + cbig_ref[...])


def kernel(edges, nodes, globals_, receivers, senders, W, b):
    E, d_edge = edges.shape
    N, d_node = nodes.shape
    d_glob = globals_.shape[-1]
    d_out = W.shape[-1]

    we = W[:d_edge]
    wcat = W[d_edge:d_edge + 2 * d_node]
    wcat = jnp.concatenate(
        [wcat[:d_node], wcat[d_node:]], axis=1)
    wg = W[d_edge + 2 * d_node:]
    b2 = b.reshape(1, d_out)

    recv32 = receivers.astype(jnp.int32)
    send32 = senders.astype(jnp.int32)

    f32 = jnp.float32
    pr, ps, c = pl.pallas_call(
        _proj_body,
        out_shape=[jax.ShapeDtypeStruct((N, d_out), f32),
                   jax.ShapeDtypeStruct((N, d_out), f32),
                   jax.ShapeDtypeStruct((1, d_out), f32)],
    )(nodes, wcat, globals_, wg, b2)

    epw = E // _NW
    nchunk = epw // _CHUNK
    mesh = plsc.VectorSubcoreMesh(core_axis_name="c", subcore_axis_name="s")
    gather = pl.kernel(
        functools.partial(_sc_gather_body, epw, nchunk),
        mesh=mesh,
        compiler_params=pltpu.CompilerParams(use_tc_tiling_on_sc=False),
        out_type=[jax.ShapeDtypeStruct((E, d_out), f32),
                  jax.ShapeDtypeStruct((E, d_out), f32)],
        scratch_types=[
            pltpu.VMEM((_CHUNK,), jnp.int32),
            pltpu.VMEM((_CHUNK,), jnp.int32),
            pltpu.VMEM((_CHUNK, d_out), f32),
            pltpu.VMEM((_CHUNK, d_out), f32),
            pltpu.SemaphoreType.DMA,
            pltpu.SemaphoreType.DMA,
        ],
    )
    gr, gs = gather(pr, ps, recv32, send32)

    pack = 128 // d_out
    wbig = jnp.kron(jnp.eye(pack, dtype=f32), we)
    cbig = jnp.tile(c, (1, pack))
    ef = edges.reshape(E // pack, 128)
    grf = gr.reshape(E // pack, 128)
    gsf = gs.reshape(E // pack, 128)

    rows = E // pack
    br = 4000
    out = pl.pallas_call(
        _combine_body,
        grid=(rows // br,),
        in_specs=[pl.BlockSpec((br, 128), lambda i: (i, 0)),
                  pl.BlockSpec((br, 128), lambda i: (i, 0)),
                  pl.BlockSpec((br, 128), lambda i: (i, 0)),
                  pl.BlockSpec((128, 128), lambda i: (0, 0)),
                  pl.BlockSpec((1, 128), lambda i: (0, 0))],
        out_specs=pl.BlockSpec((br, 128), lambda i: (i, 0)),
        out_shape=jax.ShapeDtypeStruct((rows, 128), f32),
    )(ef, grf, gsf, wbig, cbig)
    return out.reshape(E, d_out)
